# Initial kernel scaffold; baseline (speedup 1.0000x reference)
import jax
import jax.numpy as jnp
from jax.experimental import pallas as pl


def kernel(x, edge_index, Wq, bq, Wk, bk, Wv, bv, Ws, bs, Wl, bl, gamma, beta):
    raise NotImplementedError("write your pallas kernel here")


# speedup vs baseline: 1.2491x; 1.2491x over previous
import functools
import jax
import jax.numpy as jnp
from jax.experimental import pallas as pl
from jax.experimental.pallas import tpu as pltpu

N = 10000
E = 320000
D_IN = 128
D_OUT = 128
H = 4
HD = H * D_OUT


def _proj_body(x_ref, w_ref, b_ref, q_ref, k_ref, v_ref, s_ref):
    xb = x_ref[...]
    p = jnp.dot(xb, w_ref[...], preferred_element_type=jnp.float32) + b_ref[...]
    q_ref[...] = p[:, 0 * HD:1 * HD]
    k_ref[...] = p[:, 1 * HD:2 * HD]
    v_ref[...] = p[:, 2 * HD:3 * HD]
    s_ref[...] = p[:, 3 * HD:4 * HD]


def _projections(x, Wcat, bcat):
    nb = 10
    blk = N // nb
    grid = (nb,)
    out_shapes = tuple(jax.ShapeDtypeStruct((N, HD), jnp.float32) for _ in range(4))
    out_specs = tuple(pl.BlockSpec((blk, HD), lambda i: (i, 0)) for _ in range(4))
    return pl.pallas_call(
        _proj_body,
        grid=grid,
        in_specs=[
            pl.BlockSpec((blk, D_IN), lambda i: (i, 0)),
            pl.BlockSpec((D_IN, 4 * HD), lambda i: (0, 0)),
            pl.BlockSpec((1, 4 * HD), lambda i: (0, 0)),
        ],
        out_specs=out_specs,
        out_shape=out_shapes,
    )(x, Wcat, bcat)


def _final_body(aggr_ref, s_ref, wl_ref, bl_ref, g_ref, be_ref, o_ref):
    s = s_ref[...]
    parts = []
    for h in range(H):
        num = aggr_ref[h, :, 0:D_OUT]
        den = aggr_ref[h, :, D_OUT:D_OUT + 1]
        parts.append(num / (den + 1e-16))
    x_attn = jnp.concatenate(parts, axis=-1) + s
    x_lin = jnp.dot(x_attn, wl_ref[...], preferred_element_type=jnp.float32) + bl_ref[...]
    res = x_lin + x_attn[:, :D_OUT]
    mu = jnp.mean(res, axis=-1, keepdims=True)
    var = jnp.mean((res - mu) ** 2, axis=-1, keepdims=True)
    o_ref[...] = (res - mu) * jax.lax.rsqrt(var + 1e-5) * g_ref[...] + be_ref[...]


def _final(aggrT, s, Wl, bl, gamma, beta):
    nb = 10
    blk = N // nb
    return pl.pallas_call(
        _final_body,
        grid=(nb,),
        in_specs=[
            pl.BlockSpec((H, blk, 144), lambda i: (0, i, 0)),
            pl.BlockSpec((blk, HD), lambda i: (i, 0)),
            pl.BlockSpec((HD, D_OUT), lambda i: (0, 0)),
            pl.BlockSpec((1, D_OUT), lambda i: (0, 0)),
            pl.BlockSpec((1, D_OUT), lambda i: (0, 0)),
            pl.BlockSpec((1, D_OUT), lambda i: (0, 0)),
        ],
        out_specs=pl.BlockSpec((blk, D_OUT), lambda i: (i, 0)),
        out_shape=jax.ShapeDtypeStruct((N, D_OUT), jnp.float32),
    )(aggrT, s, Wl, bl, gamma, beta)


def _edge_phase_jnp(q, k, v, src, dst):
    scale = jnp.sqrt(float(D_OUT))
    qh = q.reshape(N, H, D_OUT)
    kh = k.reshape(N, H, D_OUT)
    vh = v.reshape(N, H, D_OUT)
    alpha = jnp.sum(qh[dst] * kh[src], axis=-1) / scale
    m = jnp.full((N, H), -jnp.inf, jnp.float32).at[dst].max(alpha)
    a = jnp.exp(alpha - m[dst])
    den = jnp.zeros((N, H), jnp.float32).at[dst].add(a)
    msg = vh[src] * a[:, :, None]
    num = jnp.zeros((N, H, D_OUT), jnp.float32).at[dst].add(msg)
    aggrT = jnp.zeros((H, N, 144), jnp.float32)
    aggrT = aggrT.at[:, :, :D_OUT].set(num.transpose(1, 0, 2))
    aggrT = aggrT.at[:, :, D_OUT].set(den.T)
    return aggrT


def kernel(x, edge_index, Wq, bq, Wk, bk, Wv, bv, Ws, bs, Wl, bl, gamma, beta):
    Wcat = jnp.concatenate([Wq, Wk, Wv, Ws], axis=1)
    bcat = jnp.concatenate([bq, bk, bv, bs]).reshape(1, 4 * HD)
    q, k, v, s = _projections(x, Wcat, bcat)
    src = edge_index[0]
    dst = edge_index[1]
    aggrT = _edge_phase_jnp(q, k, v, src, dst)
    return _final(aggrT, s, Wl, bl.reshape(1, D_OUT), gamma.reshape(1, D_OUT),
                  beta.reshape(1, D_OUT))


# speedup vs baseline: 9.4003x; 7.5254x over previous
import functools
import jax
import jax.numpy as jnp
from jax import lax
from jax.experimental import pallas as pl
from jax.experimental.pallas import tpu as pltpu
from jax.experimental.pallas import tpu_sc as plsc

N = 10000
E = 320000
D_IN = 128
D_OUT = 128
H = 4
HD = H * D_OUT

NC = 2
NS = 16
NW = NC * NS
L = 16

NP = 10240
HNP = NP // 2
MTAB = NP * 4
EPT_A = E // NW
EPT_C = E // NS
AGG_W = 128
NEG = -3.0e38


def _proj_body(x_ref, w_ref, b_ref, q_ref, k_ref, v_ref, s_ref):
    xb = x_ref[...]
    p = jnp.dot(xb, w_ref[...], preferred_element_type=jnp.float32) + b_ref[...]
    q_ref[...] = p[:, 0 * HD:1 * HD]
    k_ref[...] = p[:, 1 * HD:2 * HD]
    v_ref[...] = p[:, 2 * HD:3 * HD]
    s_ref[...] = p[:, 3 * HD:4 * HD]


def _projections(x, Wcat, bcat):
    nb = 10
    blk = N // nb
    out_shapes = tuple(jax.ShapeDtypeStruct((N, HD), jnp.float32) for _ in range(4))
    out_specs = tuple(pl.BlockSpec((blk, HD), lambda i: (i, 0)) for _ in range(4))
    return pl.pallas_call(
        _proj_body,
        grid=(nb,),
        in_specs=[
            pl.BlockSpec((blk, D_IN), lambda i: (i, 0)),
            pl.BlockSpec((D_IN, 4 * HD), lambda i: (0, 0)),
            pl.BlockSpec((1, 4 * HD), lambda i: (0, 0)),
        ],
        out_specs=out_specs,
        out_shape=out_shapes,
    )(x, Wcat, bcat)


def _final_body(aggr_ref, den_ref, s_ref, wl_ref, bl_ref, g_ref, be_ref, o_ref):
    s = s_ref[...]
    parts = []
    for h in range(H):
        num = aggr_ref[h, :, :]
        den = den_ref[:, h:h + 1]
        parts.append(num / (den + 1e-16))
    x_attn = jnp.concatenate(parts, axis=-1) + s
    x_lin = jnp.dot(x_attn, wl_ref[...], preferred_element_type=jnp.float32) + bl_ref[...]
    res = x_lin + x_attn[:, :D_OUT]
    mu = jnp.mean(res, axis=-1, keepdims=True)
    var = jnp.mean((res - mu) ** 2, axis=-1, keepdims=True)
    o_ref[...] = (res - mu) * jax.lax.rsqrt(var + 1e-5) * g_ref[...] + be_ref[...]


def _final(aggrT, den, s, Wl, bl, gamma, beta):
    nb = 10
    blk = N // nb
    return pl.pallas_call(
        _final_body,
        grid=(nb,),
        in_specs=[
            pl.BlockSpec((H, blk, AGG_W), lambda i: (0, i, 0)),
            pl.BlockSpec((blk, H), lambda i: (i, 0)),
            pl.BlockSpec((blk, HD), lambda i: (i, 0)),
            pl.BlockSpec((HD, D_OUT), lambda i: (0, 0)),
            pl.BlockSpec((1, D_OUT), lambda i: (0, 0)),
            pl.BlockSpec((1, D_OUT), lambda i: (0, 0)),
            pl.BlockSpec((1, D_OUT), lambda i: (0, 0)),
        ],
        out_specs=pl.BlockSpec((blk, D_OUT), lambda i: (i, 0)),
        out_shape=jax.ShapeDtypeStruct((N, D_OUT), jnp.float32),
    )(aggrT, den, s, Wl, bl, gamma, beta)


_MESH = plsc.VectorSubcoreMesh(core_axis_name="c", subcore_axis_name="s")
_IOTA = lambda: lax.iota(jnp.int32, L)


def _take(x, i):
    dn = lax.GatherDimensionNumbers(offset_dims=(), collapsed_slice_dims=(0,),
                                    start_index_map=(0,))
    return lax.gather(x, i[:, None], dn, slice_sizes=(1,),
                      mode=lax.GatherScatterMode.PROMISE_IN_BOUNDS)


def _alpha_kernel_body(q_hbm, k_hbm, src_hbm, dst_hbm,
                       alpha_hbm, mtab_hbm,
                       qr0, qr1, kr0, kr1, si_v, di_v, aw0, aw1, mt_v,
                       semA, semB, semW):
    wid = lax.axis_index("s") * NC + lax.axis_index("c")
    ebase = wid * EPT_A
    inv_scale = 1.0 / (float(D_OUT) ** 0.5)
    C = 8
    nch = EPT_A // C
    sems = (semA, semB)
    qrs = (qr0, qr1)
    krs = (kr0, kr1)
    it = _IOTA()

    pltpu.sync_copy(src_hbm.at[pl.ds(ebase, EPT_A)], si_v)
    pltpu.sync_copy(dst_hbm.at[pl.ds(ebase, EPT_A)], di_v)

    def init_body(j, _):
        mt_v[pl.ds(j * L, L)] = jnp.full((L,), NEG, jnp.float32)
        return _
    lax.fori_loop(0, MTAB // L, init_body, None)

    def fetch(i, b):
        i = jnp.minimum(i, nch - 1)
        sl = pl.ds(i * C, C)
        pltpu.async_copy(q_hbm.at[di_v.at[sl]], qrs[b], sems[b])
        pltpu.async_copy(k_hbm.at[si_v.at[sl]], krs[b], sems[b])

    def wait(b):
        pltpu.make_async_copy(q_hbm.at[di_v.at[pl.ds(0, C)]], qrs[b], sems[b]).wait()
        pltpu.make_async_copy(k_hbm.at[si_v.at[pl.ds(0, C)]], krs[b], sems[b]).wait()

    def compute(b, lane_off, avec):
        qr, kr = qrs[b], krs[b]
        avec = list(avec)
        for e in range(C):
            for h in range(H):
                p = qr[e, pl.ds(h * D_OUT, L)] * kr[e, pl.ds(h * D_OUT, L)]
                for d in range(1, D_OUT // L):
                    off = h * D_OUT + d * L
                    p = p + qr[e, pl.ds(off, L)] * kr[e, pl.ds(off, L)]
                for sh in (8, 4, 2, 1):
                    p = p + _take(p, it ^ sh)
                avec[h] = jnp.where(it == lane_off + e, p * inv_scale, avec[h])
        return avec

    def max_update(jp, avec):
        dv = di_v[pl.ds(jp * L, L)]
        for e in range(L):
            slot = dv[e] * 4
            a4 = jnp.full((L,), NEG, jnp.float32)
            for h in range(H):
                ab = _take(avec[h], jnp.full((L,), e, jnp.int32))
                a4 = jnp.where(it == h, ab, a4)
            old = mt_v[pl.ds(slot, L)]
            mt_v[pl.ds(slot, L)] = jnp.where(it < H, jnp.maximum(old, a4), old)

    def do_pair(jp, aw):
        zero = jnp.zeros((L,), jnp.float32)
        avec = [zero, zero, zero, zero]
        c0 = jp * 2
        wait(0)
        avec = compute(0, 0, avec)
        fetch(c0 + 2, 0)
        wait(1)
        avec = compute(1, C, avec)
        fetch(c0 + 3, 1)
        for h in range(H):
            aw[h, :] = avec[h]
            pltpu.async_copy(aw.at[h],
                             alpha_hbm.at[pl.ds(h * E + ebase + jp * L, L)], semW)
        max_update(jp, avec)

    def drain_aw(aw):
        for h in range(H):
            pltpu.make_async_copy(aw.at[h], alpha_hbm.at[pl.ds(0, L)], semW).wait()

    fetch(0, 0)
    fetch(1, 1)
    npair2 = (nch // 2) // 2

    def body(j2, _):
        @pl.when(j2 > 0)
        def _drain():
            drain_aw(aw0)
            drain_aw(aw1)
        do_pair(j2 * 2, aw0)
        do_pair(j2 * 2 + 1, aw1)
        return _
    lax.fori_loop(0, npair2, body, None)
    drain_aw(aw0)
    drain_aw(aw1)
    do_pair(npair2 * 2, aw0)
    drain_aw(aw0)
    wait(0)
    wait(1)

    pltpu.sync_copy(mt_v, mtab_hbm.at[pl.ds(wid * MTAB, MTAB)])


def _alpha_max(q, k, src, dst):
    C = 8
    f = pl.kernel(
        _alpha_kernel_body,
        mesh=_MESH,
        out_type=[
            jax.ShapeDtypeStruct((H * E,), jnp.float32),
            jax.ShapeDtypeStruct((NW * MTAB,), jnp.float32),
        ],
        scratch_types=[
            pltpu.VMEM((C, HD), jnp.float32), pltpu.VMEM((C, HD), jnp.float32),
            pltpu.VMEM((C, HD), jnp.float32), pltpu.VMEM((C, HD), jnp.float32),
            pltpu.VMEM((EPT_A,), jnp.int32), pltpu.VMEM((EPT_A,), jnp.int32),
            pltpu.VMEM((H, L), jnp.float32), pltpu.VMEM((H, L), jnp.float32),
            pltpu.VMEM((MTAB,), jnp.float32),
            pltpu.SemaphoreType.DMA, pltpu.SemaphoreType.DMA,
            pltpu.SemaphoreType.DMA,
        ],
    )
    return f(q, k, src, dst)


def _merge_body(op, mtab_hbm, mfin_hbm, buf_v, out_v, sem):
    wid = lax.axis_index("s") * NC + lax.axis_index("c")
    span = MTAB // NW
    for t in range(NW):
        pltpu.async_copy(mtab_hbm.at[pl.ds(t * MTAB + wid * span, span)],
                         buf_v.at[pl.ds(t * span, span)], sem)
    for t in range(NW):
        pltpu.make_async_copy(mtab_hbm.at[pl.ds(t * MTAB + wid * span, span)],
                              buf_v.at[pl.ds(t * span, span)], sem).wait()

    def body(g, _):
        acc = buf_v[pl.ds(g * L, L)]
        for t in range(1, NW):
            acc = op(acc, buf_v[pl.ds(t * span + g * L, L)])
        out_v[pl.ds(g * L, L)] = acc
        return _
    lax.fori_loop(0, span // L, body, None)
    pltpu.sync_copy(out_v, mfin_hbm.at[pl.ds(wid * span, span)])


def _merge(mtab, op):
    f = pl.kernel(
        functools.partial(_merge_body, op),
        mesh=_MESH,
        out_type=[jax.ShapeDtypeStruct((MTAB,), jnp.float32)],
        scratch_types=[
            pltpu.VMEM((MTAB,), jnp.float32),
            pltpu.VMEM((MTAB // NW,), jnp.float32),
            pltpu.SemaphoreType.DMA,
        ],
    )
    return f(mtab)


def _aggr_body(vflat_hbm, src_hbm, dst_hbm, alpha_hbm, mfin_hbm,
               aggr_hbm,
               sidx_v, didx_v, al_v,
               vr0, vr1, sb0, sb1, mg0, mg1, tab_sh, semA, semB):
    c = lax.axis_index("c")
    s = lax.axis_index("s")
    ebase = s * EPT_C
    nch = EPT_C // L
    sems = (semA, semB)
    vrs = (vr0, vr1)
    sbs = (sb0, sb1)
    mgs = (mg0, mg1)
    it = _IOTA()

    pltpu.sync_copy(src_hbm.at[pl.ds(ebase, EPT_C)], sidx_v)
    pltpu.sync_copy(dst_hbm.at[pl.ds(ebase, EPT_C)], didx_v)

    nreal = HNP // NS
    ndump = 256 // NS

    for r in range(4):
        h = c * 2 + r // 2
        nbase = (r % 2) * HNP

        for e in range(L):
            for d in range(AGG_W // L):
                sb0[e, pl.ds(d * L, L)] = jnp.zeros((L,), jnp.float32)

        def zero_body(j, _):
            pltpu.sync_copy(sb0, tab_sh.at[pl.ds(s * nreal + j * L, L)])
            return _
        lax.fori_loop(0, nreal // L, zero_body, None)
        pltpu.sync_copy(sb0, tab_sh.at[pl.ds(HNP + s * ndump, ndump)])
        plsc.subcore_barrier()

        if r % 2 == 0:
            pltpu.sync_copy(alpha_hbm.at[pl.ds(h * E + ebase, EPT_C)], al_v)

        def fetch(i, b):
            i = jnp.minimum(i, nch - 1)
            sl = pl.ds(i * L, L)
            vslot = sidx_v[sl] * H + h
            mslot = didx_v[sl] * H + h
            pltpu.async_copy(vflat_hbm.at[vslot], vrs[b], sems[b])
            pltpu.async_copy(mfin_hbm.at[mslot], mgs[b], sems[b])

        def wait(b):
            zi = jnp.zeros((L,), jnp.int32)
            pltpu.make_async_copy(vflat_hbm.at[zi], vrs[b], sems[b]).wait()
            pltpu.make_async_copy(mfin_hbm.at[zi], mgs[b], sems[b]).wait()

        def compute(i, b):
            vr, sb, mg = vrs[b], sbs[b], mgs[b]
            sl = pl.ds(i * L, L)
            w = jnp.exp(al_v[sl] - mg[...])
            dv = didx_v[sl]
            for e in range(L):
                we = _take(w, jnp.full((L,), e, jnp.int32))
                for d in range(D_OUT // L):
                    sb[e, pl.ds(d * L, L)] = vr[e, pl.ds(d * L, L)] * we
            in_half = (dv >= nbase) & (dv < nbase + HNP)
            tidx = jnp.where(in_half, dv - nbase, HNP + (dv & 255))
            pltpu.sync_copy(sb, tab_sh.at[tidx], add=True)

        fetch(0, 0)
        fetch(1, 1)

        def body(jp, _):
            i0 = jp * 2
            wait(0)
            compute(i0, 0)
            fetch(i0 + 2, 0)
            wait(1)
            compute(i0 + 1, 1)
            fetch(i0 + 3, 1)
            return _
        lax.fori_loop(0, nch // 2, body, None)
        wait(0)
        wait(1)
        plsc.subcore_barrier()
        pltpu.sync_copy(tab_sh.at[pl.ds(s * nreal, nreal)],
                        aggr_hbm.at[pl.ds(h * NP + nbase + s * nreal, nreal)])


def _aggregate(vflat, src, dst, alpha, mfin):
    f = pl.kernel(
        _aggr_body,
        mesh=_MESH,
        out_type=[jax.ShapeDtypeStruct((H * NP, AGG_W), jnp.float32)],
        scratch_types=[
            pltpu.VMEM((EPT_C,), jnp.int32), pltpu.VMEM((EPT_C,), jnp.int32),
            pltpu.VMEM((EPT_C,), jnp.float32),
            pltpu.VMEM((L, D_OUT), jnp.float32), pltpu.VMEM((L, D_OUT), jnp.float32),
            pltpu.VMEM((L, AGG_W), jnp.float32), pltpu.VMEM((L, AGG_W), jnp.float32),
            pltpu.VMEM((L,), jnp.float32), pltpu.VMEM((L,), jnp.float32),
            pltpu.VMEM_SHARED((HNP + 256, AGG_W), jnp.float32),
            pltpu.SemaphoreType.DMA, pltpu.SemaphoreType.DMA,
        ],
    )
    return f(vflat, src, dst, alpha, mfin)


def _denom_body(dst_hbm, alpha_hbm, mfin_hbm,
                dtab_hbm,
                di_v, al4_v, mgA, mgB, dt_v, semA, semB):
    wid = lax.axis_index("s") * NC + lax.axis_index("c")
    ebase = wid * EPT_A
    nch = EPT_A // L
    sems = (semA, semB)
    mgs = (mgA, mgB)
    it = _IOTA()

    pltpu.sync_copy(dst_hbm.at[pl.ds(ebase, EPT_A)], di_v)
    for h in range(H):
        pltpu.sync_copy(alpha_hbm.at[pl.ds(h * E + ebase, EPT_A)],
                        al4_v.at[pl.ds(h * EPT_A, EPT_A)])

    def dinit_body(j, _):
        dt_v[pl.ds(j * L, L)] = jnp.zeros((L,), jnp.float32)
        return _
    lax.fori_loop(0, MTAB // L, dinit_body, None)

    def fetch(i, b):
        i = jnp.minimum(i, nch - 1)
        dv = di_v[pl.ds(i * L, L)]
        for h in range(H):
            pltpu.async_copy(mfin_hbm.at[dv * H + h],
                             mgs[b].at[pl.ds(h * L, L)], sems[b])

    def wait(b):
        zi = jnp.zeros((L,), jnp.int32)
        for h in range(H):
            pltpu.make_async_copy(mfin_hbm.at[zi],
                                  mgs[b].at[pl.ds(h * L, L)], sems[b]).wait()

    def compute(i, b):
        mg = mgs[b]
        sl = pl.ds(i * L, L)
        dv = di_v[sl]
        ws = [jnp.exp(al4_v[pl.ds(h * EPT_A + i * L, L)] - mg[pl.ds(h * L, L)])
              for h in range(H)]
        for e in range(L):
            w4 = jnp.zeros((L,), jnp.float32)
            for h in range(H):
                wb = _take(ws[h], jnp.full((L,), e, jnp.int32))
                w4 = jnp.where(it == h, wb, w4)
            slot = dv[e] * 4
            old = dt_v[pl.ds(slot, L)]
            dt_v[pl.ds(slot, L)] = jnp.where(it < H, old + w4, old)

    fetch(0, 0)
    fetch(1, 1)

    def body(jp, _):
        i0 = jp * 2
        wait(0)
        compute(i0, 0)
        fetch(i0 + 2, 0)
        wait(1)
        compute(i0 + 1, 1)
        fetch(i0 + 3, 1)
        return _
    lax.fori_loop(0, nch // 2, body, None)
    wait(0)
    wait(1)
    pltpu.sync_copy(dt_v, dtab_hbm.at[pl.ds(wid * MTAB, MTAB)])


def _denom(dst, alpha, mfin):
    f = pl.kernel(
        _denom_body,
        mesh=_MESH,
        out_type=[jax.ShapeDtypeStruct((NW * MTAB,), jnp.float32)],
        scratch_types=[
            pltpu.VMEM((EPT_A,), jnp.int32),
            pltpu.VMEM((H * EPT_A,), jnp.float32),
            pltpu.VMEM((H * L,), jnp.float32), pltpu.VMEM((H * L,), jnp.float32),
            pltpu.VMEM((MTAB,), jnp.float32),
            pltpu.SemaphoreType.DMA, pltpu.SemaphoreType.DMA,
        ],
    )
    return f(dst, alpha, mfin)


def kernel(x, edge_index, Wq, bq, Wk, bk, Wv, bv, Ws, bs, Wl, bl, gamma, beta):
    Wcat = jnp.concatenate([Wq, Wk, Wv, Ws], axis=1)
    bcat = jnp.concatenate([bq, bk, bv, bs]).reshape(1, 4 * HD)
    q, k, v, s = _projections(x, Wcat, bcat)
    src = jnp.asarray(edge_index[0], jnp.int32)
    dst = jnp.asarray(edge_index[1], jnp.int32)
    alpha, mtab = _alpha_max(q, k, src, dst)
    mfin = _merge(mtab, jnp.maximum)
    if isinstance(mfin, (list, tuple)):
        mfin = mfin[0]
    vflat = v.reshape(N * H, D_OUT)
    aggrT = _aggregate(vflat, src, dst, alpha, mfin)
    if isinstance(aggrT, (list, tuple)):
        aggrT = aggrT[0]
    dtab = _denom(dst, alpha, mfin)
    if isinstance(dtab, (list, tuple)):
        dtab = dtab[0]
    den = _merge(dtab, jnp.add)
    if isinstance(den, (list, tuple)):
        den = den[0]
    aggrT = aggrT.reshape(H, NP, AGG_W)
    den = den.reshape(NP, H)[:N]
    return _final(aggrT, den, s, Wl, bl.reshape(1, D_OUT), gamma.reshape(1, D_OUT),
                  beta.reshape(1, D_OUT))
